# baseline (device time: 19855 ns/iter reference)
import jax
import jax.numpy as jnp
from jax import lax
from jax.experimental import pallas as pl
from jax.experimental.pallas import tpu as pltpu

N_DEV = 4


def kernel(partial, resid, gamma):
    x = partial.reshape(partial.shape[-2], partial.shape[-1])
    m, n = x.shape
    half = m // 2
    quart = m // 4
    eighth = m // 8
    th32 = m // 16
    gamma2d = gamma.reshape(1, n)

    def body(x_ref, resid_hbm, gamma_ref, out_ref,
             resid_v, rA1, rB1, rA2, rB2, send_sems, recv_sems, copy_sem):
        my = lax.axis_index("i")
        pa = my ^ 1
        pb = 3 - my

        kA1 = (my ^ (my >> 1)) & 1
        kA2 = my >> 1
        kB1 = my >> 1
        kB2 = my & 1

        A_keep1 = kA1 * quart
        A_send1 = (1 - kA1) * quart
        fwd2A = A_keep1 + (1 - kA2) * eighth
        own2A = A_keep1 + kA2 * eighth
        c0A = A_send1 + (1 - kA2) * eighth
        c1A = A_send1 + kA2 * eighth

        B_keep1 = half + kB1 * quart
        B_send1 = half + (1 - kB1) * quart
        fwd2B = B_keep1 + (1 - kB2) * eighth
        own2B = B_keep1 + kB2 * eighth
        c0B = B_send1 + kB2 * eighth
        c1B = B_send1 + (1 - kB2) * eighth

        cp = pltpu.make_async_copy(resid_hbm, resid_v, copy_sem)
        cp.start()

        barrier_sem = pltpu.get_barrier_semaphore()
        for nbr in [pa, pb]:
            pl.semaphore_signal(
                barrier_sem, inc=1,
                device_id=(nbr,), device_id_type=pl.DeviceIdType.MESH,
            )
        pl.semaphore_wait(barrier_sem, 2)

        def rc(src_ref, src_start, rows, dst_ref, dst_start, peer, idx):
            return pltpu.make_async_remote_copy(
                src_ref=src_ref.at[pl.ds(src_start, rows), :],
                dst_ref=dst_ref.at[pl.ds(dst_start, rows), :],
                send_sem=send_sems.at[idx],
                recv_sem=recv_sems.at[idx],
                device_id=(peer,),
                device_id_type=pl.DeviceIdType.MESH,
            )

        s1a0 = rc(x_ref, c0A, eighth, rA1, 0, pa, 0)
        s1a1 = rc(x_ref, c1A, eighth, rA1, eighth, pa, 1)
        s1b0 = rc(x_ref, c0B, eighth, rB1, 0, pb, 2)
        s1b1 = rc(x_ref, c1B, eighth, rB1, eighth, pb, 3)
        s1a0.start()
        s1a1.start()
        s1b0.start()
        s1b1.start()

        s1a0.wait_recv()
        out_ref[pl.ds(fwd2A, eighth), :] = (
            x_ref[pl.ds(fwd2A, eighth), :] + rA1[pl.ds(0, eighth), :]
        )
        s2a0 = rc(out_ref, fwd2A, th32, rA2, 0, pb, 4)
        s2a1 = rc(out_ref, fwd2A + th32, th32, rA2, th32, pb, 5)
        s2a0.start()
        s2a1.start()

        s1b0.wait_recv()
        out_ref[pl.ds(fwd2B, eighth), :] = (
            x_ref[pl.ds(fwd2B, eighth), :] + rB1[pl.ds(0, eighth), :]
        )
        s2b0 = rc(out_ref, fwd2B, th32, rB2, 0, pa, 6)
        s2b1 = rc(out_ref, fwd2B + th32, th32, rB2, th32, pa, 7)
        s2b0.start()
        s2b1.start()

        s1a1.wait_recv()
        out_ref[pl.ds(own2A, eighth), :] = (
            x_ref[pl.ds(own2A, eighth), :] + rA1[pl.ds(eighth, eighth), :]
        )
        s1b1.wait_recv()
        out_ref[pl.ds(own2B, eighth), :] = (
            x_ref[pl.ds(own2B, eighth), :] + rB1[pl.ds(eighth, eighth), :]
        )

        cp.wait()
        g = gamma_ref[0, :][None, :]

        def ln_and_gather(rdma_in, start, rbuf, roff, p3, p4, i3, i4):
            rdma_in.wait_recv()
            y = (
                out_ref[pl.ds(start, th32), :]
                + rbuf[pl.ds(roff, th32), :]
                + resid_v[pl.ds(start, th32), :]
            )
            rms = jnp.sqrt(jnp.mean(y * y, axis=-1, keepdims=True) + 1e-6)
            out_ref[pl.ds(start, th32), :] = y / rms * g
            g3 = rc(out_ref, start, th32, out_ref, start, p3, i3)
            g4 = rc(out_ref, start, th32, out_ref, start, p4, i4)
            g3.start()
            g4.start()
            return g3, g4

        g3a0, g4aa0 = ln_and_gather(s2a0, own2A, rA2, 0, pb, pa, 8, 12)
        g3b0, g4ab0 = ln_and_gather(s2b0, own2B, rB2, 0, pa, pb, 10, 14)
        g3a1, g4aa1 = ln_and_gather(s2a1, own2A + th32, rA2, th32, pb, pa, 9, 13)
        g3b1, g4ab1 = ln_and_gather(s2b1, own2B + th32, rB2, th32, pa, pb, 11, 15)

        g3a0.wait_recv()
        g4ba0 = rc(out_ref, fwd2A, th32, out_ref, fwd2A, pa, 16)
        g4ba0.start()
        g3b0.wait_recv()
        g4bb0 = rc(out_ref, fwd2B, th32, out_ref, fwd2B, pb, 18)
        g4bb0.start()
        g3a1.wait_recv()
        g4ba1 = rc(out_ref, fwd2A + th32, th32, out_ref, fwd2A + th32, pa, 17)
        g4ba1.start()
        g3b1.wait_recv()
        g4bb1 = rc(out_ref, fwd2B + th32, th32, out_ref, fwd2B + th32, pb, 19)
        g4bb1.start()

        for r in (g4aa0, g4aa1, g4ab0, g4ab1, g4ba0, g4ba1, g4bb0, g4bb1):
            r.wait_recv()

        for r in (s1a0, s1a1, s1b0, s1b1, s2a0, s2a1, s2b0, s2b1,
                  g3a0, g3a1, g3b0, g3b1, g4aa0, g4aa1, g4ab0, g4ab1,
                  g4ba0, g4ba1, g4bb0, g4bb1):
            r.wait_send()

    return pl.pallas_call(
        body,
        out_shape=jax.ShapeDtypeStruct((m, n), jnp.float32),
        in_specs=[
            pl.BlockSpec(memory_space=pltpu.VMEM),
            pl.BlockSpec(memory_space=pl.ANY),
            pl.BlockSpec(memory_space=pltpu.VMEM),
        ],
        out_specs=pl.BlockSpec(memory_space=pltpu.VMEM),
        scratch_shapes=[
            pltpu.VMEM((m, n), jnp.float32),
            pltpu.VMEM((quart, n), jnp.float32),
            pltpu.VMEM((quart, n), jnp.float32),
            pltpu.VMEM((eighth, n), jnp.float32),
            pltpu.VMEM((eighth, n), jnp.float32),
            pltpu.SemaphoreType.DMA((20,)),
            pltpu.SemaphoreType.DMA((20,)),
            pltpu.SemaphoreType.DMA,
        ],
        compiler_params=pltpu.CompilerParams(collective_id=0),
    )(x, resid, gamma2d)
